# Initial kernel scaffold; baseline (speedup 1.0000x reference)
import jax
import jax.numpy as jnp
from jax.experimental import pallas as pl


def kernel(x):
    raise NotImplementedError("write your pallas kernel here")


# speedup vs baseline: 2152.2893x; 2152.2893x over previous
import numpy as np
import jax
import jax.numpy as jnp
from jax.experimental import pallas as pl
from jax.experimental.pallas import tpu as pltpu

_REG = 0.1
_N = 2048
_ROWS = 256
_OUTS = 8
_TILE = 128
_NTILES = _N // _TILE

_W = np.asarray(np.arange(_N, 0, -1, dtype=np.float32) / np.float32(_REG),
                dtype=np.float32).reshape(_N, 1)


def _body(xT_ref, w_ref, out_ref, z_s, scan_s):
    x = xT_ref[...]

    idx = jax.lax.broadcasted_iota(jnp.int32, (_N, 1), 0)
    for p in range(11):
        for q in range(p, -1, -1):
            d = 1 << q
            lower = (idx & d) == 0
            up = (idx & (1 << (p + 1))) == 0
            partner = jnp.where(lower, jnp.roll(x, -d, axis=0),
                                jnp.roll(x, d, axis=0))
            take_min = lower == up
            x = jnp.where(take_min, jnp.minimum(x, partner),
                          jnp.maximum(x, partner))

    w = w_ref[...]
    z = x + w
    z_s[...] = z.reshape(_NTILES, _TILE, _ROWS)

    def inner(i, acc):
        acc = acc + z_s[:, i, :]
        scan_s[:, pl.ds(i, 1), :] = acc[:, None, :]
        return acc

    jax.lax.fori_loop(0, _TILE, inner,
                      jnp.zeros((_NTILES, _ROWS), jnp.float32))

    scan = scan_s[...]
    totals = scan[:, _TILE - 1, :]
    carry_rows = [jnp.zeros((1, _ROWS), jnp.float32)]
    acc2 = totals[0:1]
    for t in range(1, _NTILES):
        carry_rows.append(acc2)
        acc2 = acc2 + totals[t:t + 1]
    carry = jnp.concatenate(carry_rows, axis=0)
    P = (scan + carry[:, None, :]).reshape(_N, _ROWS)

    Pj = jnp.concatenate([jnp.zeros((1, _ROWS), jnp.float32), P[:_N - 1]],
                         axis=0)

    jvec = jax.lax.broadcasted_iota(jnp.int32, (_N, 1), 0)
    mrun = None
    inf = jnp.float32(np.inf)
    for t in range(_OUTS):
        k = _N - 1 - t
        den = (k + 1 - jvec).astype(jnp.float32)
        slope = (P[k:k + 1, :] - Pj) / den
        mrun = slope if t == 0 else jnp.maximum(mrun, slope)
        masked = jnp.where(jvec <= k, mrun, inf)
        v = jnp.min(masked, axis=0)
        out_ref[t, :] = v - w_ref[k, 0]


def kernel(x):
    B, S, N = x.shape
    xT = jnp.transpose(x.reshape(B * S, N))
    out = pl.pallas_call(
        _body,
        out_shape=jax.ShapeDtypeStruct((_OUTS, _ROWS), jnp.float32),
        scratch_shapes=[
            pltpu.VMEM((_NTILES, _TILE, _ROWS), jnp.float32),
            pltpu.VMEM((_NTILES, _TILE, _ROWS), jnp.float32),
        ],
    )(xT, jnp.asarray(_W))
    return jnp.transpose(out).reshape(B, S, _OUTS)
